# baseline (speedup 1.0000x reference)
import functools

import jax
import jax.numpy as jnp
from jax.experimental import pallas as pl
from jax.experimental.pallas import tpu as pltpu


def _body(x_ref, w_ref, el_ref, diff_ref, lsn_ref, out_ref, loss_ref, *, M, N, F):
    accs = [None] * N
    for m in range(M):
        xv = x_ref[:, m * F : (m + 1) * F]
        for n in range(N):
            t = w_ref[:, m * N + n : m * N + n + 1] * xv
            accs[n] = t if accs[n] is None else accs[n] + t
    for n in range(N):
        out_ref[:, n * F : (n + 1) * F] = accs[n]
    for n in range(N):
        col = el_ref[:, n : n + 1]
        for m in range(M):
            col = col + (
                w_ref[:, m * N + n : m * N + n + 1] * diff_ref[m, n] + lsn_ref[m, n]
            )
        loss_ref[:, n : n + 1] = col


def kernel(x, extra_loss, logits):
    B, M, F = x.shape
    N = logits.shape[1]
    probs = jax.nn.sigmoid(logits)
    w = jax.random.bernoulli(jax.random.key(42), probs, shape=(B, M, N)).astype(
        jnp.float32
    )
    ls = jax.nn.log_sigmoid(logits)
    lsn = jax.nn.log_sigmoid(-logits)
    diff = ls - lsn

    xf = x.reshape(B, M * F)
    wf = w.reshape(B, M * N)

    bB = 16
    grid = (B // bB,)
    out_shapes = (
        jax.ShapeDtypeStruct((B, N * F), jnp.float32),
        jax.ShapeDtypeStruct((B, N), jnp.float32),
    )
    fn = pl.pallas_call(
        functools.partial(_body, M=M, N=N, F=F),
        grid=grid,
        in_specs=[
            pl.BlockSpec((bB, M * F), lambda i: (i, 0)),
            pl.BlockSpec((bB, M * N), lambda i: (i, 0)),
            pl.BlockSpec((bB, N), lambda i: (i, 0)),
            pl.BlockSpec(memory_space=pltpu.SMEM),
            pl.BlockSpec(memory_space=pltpu.SMEM),
        ],
        out_specs=(
            pl.BlockSpec((bB, N * F), lambda i: (i, 0)),
            pl.BlockSpec((bB, N), lambda i: (i, 0)),
        ),
        out_shape=out_shapes,
        compiler_params=pltpu.CompilerParams(
            dimension_semantics=("arbitrary",),
        ),
    )
    out, loss = fn(xf, wf, extra_loss, diff, lsn)
    return (out.reshape(B, N, F), loss)
